# baseline (speedup 1.0000x reference)
import functools

import jax
import jax.numpy as jnp
from jax import lax
from jax.experimental import pallas as pl
from jax.experimental.pallas import tpu as pltpu
from jax.experimental.pallas import tpu_sc as plsc

H = 224
W = 224
HW = H * W
SENT = HW
HALF = HW // 2
DBUF = HALF + 16
NROW = 400
KDIM = 375
VPR = 125
RC = 80


def _project_body(vf_ref, rot_ref, trans_ref, intr_ref, pix_ref, dep_ref):
    def rb(s):
        return s.astype(jnp.bfloat16).astype(jnp.float32)

    ii = lax.broadcasted_iota(jnp.int32, (KDIM, 384), 0)
    jj = lax.broadcasted_iota(jnp.int32, (KDIM, 384), 1)
    c_of_i = ii // 3
    comp = ii - c_of_i * 3
    cj = jnp.bitwise_and(jj, 127)
    gj = jj >> 7
    match = c_of_i == cj
    coeff = jnp.zeros((KDIM, 384), jnp.float32)
    for g in range(3):
        for l in range(3):
            coeff = jnp.where(match & (gj == g) & (comp == l),
                              rb(rot_ref[0, g, l]), coeff)

    vb = vf_ref[...].astype(jnp.bfloat16)
    out = lax.dot_general(vb, coeff.astype(jnp.bfloat16),
                          (((1,), (0,)), ((), ())),
                          preferred_element_type=jnp.float32)
    X = out[:, 0:128]
    Y = out[:, 128:256]
    Z = out[:, 256:384]

    tx = trans_ref[0, 0, 0]
    ty = trans_ref[0, 0, 1]
    tz = trans_ref[0, 0, 2]
    fx = intr_ref[0, 0, 0]
    fy = intr_ref[0, 1, 1]
    cx = intr_ref[0, 0, 2]
    cy = intr_ref[0, 1, 2]

    X = X + tx
    Y = Y + ty
    Z = Z + tz
    Zs = Z + 1e-8
    u = fx * (X / Zs) + cx
    v = fy * (Y / Zs) + cy
    u_i = u.astype(jnp.int32)
    v_i = v.astype(jnp.int32)
    col = lax.broadcasted_iota(jnp.int32, (NROW, 128), 1)
    valid = ((u_i >= 0) & (u_i < W) & (v_i >= 0) & (v_i < H)
             & (col < VPR))
    pix_ref[...] = jnp.where(valid, v_i * W + u_i, SENT)
    dep_ref[...] = Z


def _tc_project(vflat, rotation, translation, intrinsics):
    B = rotation.shape[0]
    out_shape = (
        jax.ShapeDtypeStruct((B, NROW, 128), jnp.int32),
        jax.ShapeDtypeStruct((B, NROW, 128), jnp.float32),
    )
    return pl.pallas_call(
        _project_body,
        grid=(B,),
        in_specs=[
            pl.BlockSpec((None, NROW, KDIM), lambda b: (b, 0, 0)),
            pl.BlockSpec((1, 3, 3), lambda b: (b, 0, 0),
                         memory_space=pltpu.SMEM),
            pl.BlockSpec((1, 1, 3), lambda b: (b, 0, 0),
                         memory_space=pltpu.SMEM),
            pl.BlockSpec((1, 3, 3), lambda b: (b, 0, 0),
                         memory_space=pltpu.SMEM),
        ],
        out_specs=[
            pl.BlockSpec((None, NROW, 128), lambda b: (b, 0, 0)),
            pl.BlockSpec((None, NROW, 128), lambda b: (b, 0, 0)),
        ],
        out_shape=out_shape,
    )(vflat, rotation, translation, intrinsics)


def _sc_scatter(pix, dep, B):
    n_chunks = NROW // RC
    mesh = plsc.VectorSubcoreMesh(core_axis_name="c", subcore_axis_name="s")

    @functools.partial(
        pl.kernel,
        mesh=mesh,
        out_type=jax.ShapeDtypeStruct((B * HW,), jnp.float32),
        compiler_params=pltpu.CompilerParams(needs_layout_passes=False),
        scratch_types=[
            pltpu.VMEM((DBUF,), jnp.float32),
            pltpu.VMEM((RC, 128), jnp.int32),
            pltpu.VMEM((RC, 128), jnp.float32),
            pltpu.VMEM((RC, 128), jnp.int32),
            pltpu.VMEM((RC, 128), jnp.float32),
            pltpu.SemaphoreType.DMA,
            pltpu.SemaphoreType.DMA,
        ],
    )
    def scatter_kernel(pix_hbm, dep_hbm, out_hbm, dbuf,
                       pixv0, depv0, pixv1, depv1, sem0, sem1):
        cid = lax.axis_index("c")
        sid = lax.axis_index("s")
        b = sid
        lo = cid * HALF
        zeros = jnp.zeros((16,), jnp.float32)

        def zero_body(j, carry):
            dbuf[pl.ds(j * 16, 16)] = zeros
            return carry

        lax.fori_loop(0, DBUF // 16, zero_body, 0, unroll=8)

        bufs = ((pixv0, depv0, sem0), (pixv1, depv1, sem1))

        def start(g):
            pv, dv, sm = bufs[g % 2]
            r0 = pl.multiple_of(g * RC, 8)
            d1 = pltpu.async_copy(pix_hbm.at[b, pl.ds(r0, RC)], pv, sm)
            d2 = pltpu.async_copy(dep_hbm.at[b, pl.ds(r0, RC)], dv, sm)
            return d1, d2

        descs = start(0)
        for g in range(n_chunks):
            d1, d2 = descs
            d1.wait()
            d2.wait()
            if g + 1 < n_chunks:
                descs = start(g + 1)
            pv, dv, _ = bufs[g % 2]

            def row_body(r, carry, pv=pv, dv=dv):
                for v8 in range(8):
                    p = pv[r, pl.ds(v8 * 16, 16)]
                    d = dv[r, pl.ds(v8 * 16, 16)]
                    p_loc = p - lo
                    m = p_loc.astype(jnp.uint32) < jnp.uint32(HALF)
                    plsc.store_scatter(dbuf, [p_loc], d, mask=m)
                return carry

            lax.fori_loop(0, RC, row_body, 0, unroll=2)

        out_off = pl.multiple_of(b * HW + lo, 8)
        pltpu.sync_copy(dbuf.at[pl.ds(0, HALF)],
                        out_hbm.at[pl.ds(out_off, HALF)])

    return scatter_kernel(pix, dep)


def kernel(vertices, rotation, translation, camera_intrinsics):
    B, N, _ = vertices.shape
    vflat = vertices.reshape(B, NROW, KDIM)
    pix, dep = _tc_project(vflat, rotation, translation.reshape(B, 1, 3),
                           camera_intrinsics)
    flat = _sc_scatter(pix, dep, B)
    return flat.reshape(B, 1, H, W)
